# baseline (speedup 1.0000x reference)
import functools

import jax
import jax.numpy as jnp
from jax import lax
from jax.experimental import pallas as pl
from jax.experimental.pallas import tpu as pltpu
from jax.experimental.pallas import tpu_sc as plsc

N = 10000
E = 320000
D = 128
NP = 10240
NC = 2
NS = 16
NW = NC * NS
EPW = E // NW
CH = 100
NSUP = EPW // CH
SLP = NP // NS
NIB = 3

_MESH = plsc.VectorSubcoreMesh(core_axis_name="c", subcore_axis_name="s")
_SC_PARAMS = pltpu.CompilerParams(needs_layout_passes=False)


@functools.partial(
    pl.kernel,
    out_type=jax.ShapeDtypeStruct((NC, NP), jnp.float32),
    mesh=_MESH,
    compiler_params=_SC_PARAMS,
    scratch_types=[
        pltpu.VMEM((EPW,), jnp.int32),
        pltpu.VMEM((NP,), jnp.float32),
        pltpu.VMEM((NS, SLP), jnp.float32),
        pltpu.VMEM((SLP,), jnp.float32),
        pltpu.VMEM_SHARED((NS, NP), jnp.float32),
    ],
)
def _deg_hist(dst_hbm, out_hbm, dsts, hist, buf, comb, hist_all):
    cid = lax.axis_index("c")
    sid = lax.axis_index("s")
    wid = sid * NC + cid
    z16 = jnp.zeros((16,), jnp.float32)
    ones16 = jnp.ones((16,), jnp.float32)

    def zloop(i, _):
        hist[pl.ds(i * 16, 16)] = z16
        return 0

    lax.fori_loop(0, NP // 16, zloop, 0)
    pltpu.sync_copy(dst_hbm.at[pl.ds(wid * EPW, EPW)], dsts)

    def hloop(i, _):
        idx = dsts[pl.ds(i * 16, 16)]
        plsc.addupdate_scatter(hist, [idx], ones16)
        return 0

    lax.fori_loop(0, EPW // 16, hloop, 0)
    pltpu.sync_copy(hist, hist_all.at[sid])
    plsc.subcore_barrier()
    pltpu.sync_copy(hist_all.at[pl.ds(0, NS), pl.ds(sid * SLP, SLP)], buf)

    def cloop(k, _):
        v = buf[0, pl.ds(k * 16, 16)]
        for r in range(1, NS):
            v = v + buf[r, pl.ds(k * 16, 16)]
        comb[pl.ds(k * 16, 16)] = v
        return 0

    lax.fori_loop(0, SLP // 16, cloop, 0)
    pltpu.sync_copy(comb, out_hbm.at[cid, pl.ds(sid * SLP, SLP)])


@functools.partial(
    pl.kernel,
    out_type=jax.ShapeDtypeStruct((NC, NP, D), jnp.float32),
    mesh=_MESH,
    compiler_params=_SC_PARAMS,
    scratch_types=[
        pltpu.VMEM((NIB, CH), jnp.int32),
        pltpu.VMEM((NIB, CH), jnp.int32),
        pltpu.VMEM((NIB, CH, D), jnp.float32),
        pltpu.VMEM_SHARED((NP, D), jnp.float32),
        pltpu.SemaphoreType.DMA,
        pltpu.SemaphoreType.DMA((2,)),
        pltpu.SemaphoreType.DMA,
    ],
)
def _edge_scatter(src_hbm, dst_hbm, g_hbm, z_hbm, out_hbm, sidx, didx, rows,
                  acc_sh, gsem, ssem, isem):
    cid = lax.axis_index("c")
    sid = lax.axis_index("s")
    wid = sid * NC + cid
    base = sid * SLP

    def fire_g(slot, buf):
        pltpu.async_copy(g_hbm.at[sidx.at[slot]], rows.at[buf], gsem)

    def drain_g(slot, buf):
        pltpu.make_async_copy(g_hbm.at[sidx.at[slot]], rows.at[buf],
                              gsem).wait()

    def fire_s(slot, buf, r):
        pltpu.async_copy(rows.at[buf], acc_sh.at[didx.at[slot]], ssem.at[r],
                         add=True)

    def drain_s(slot, buf, r):
        pltpu.make_async_copy(rows.at[buf], acc_sh.at[didx.at[slot]],
                              ssem.at[r]).wait()

    def fire_idx(m, slot):
        pltpu.async_copy(src_hbm.at[wid, m], sidx.at[slot], isem)
        pltpu.async_copy(dst_hbm.at[wid, m], didx.at[slot], isem)

    def drain_idx(m, slot):
        pltpu.make_async_copy(src_hbm.at[wid, m], sidx.at[slot], isem).wait()
        pltpu.make_async_copy(dst_hbm.at[wid, m], didx.at[slot], isem).wait()

    pltpu.sync_copy(src_hbm.at[wid, 0], sidx.at[0])
    pltpu.sync_copy(dst_hbm.at[wid, 0], didx.at[0])
    fire_idx(1, 1)
    fire_g(0, 0)
    pltpu.sync_copy(z_hbm, acc_sh.at[pl.ds(base, SLP)])
    plsc.subcore_barrier()

    drain_g(0, 0)
    fire_s(0, 0, 0)
    drain_idx(1, 1)
    fire_idx(2, 2)
    fire_g(1, 1)

    def step(m, _):
        p = lax.rem(m, NIB)
        pm = lax.rem(m + NIB - 1, NIB)
        p1 = lax.rem(m + 1, NIB)
        p2 = lax.rem(m + 2, NIB)
        r = lax.rem(m, 2)
        rm = lax.rem(m + 1, 2)
        drain_g(p, p)
        fire_s(p, p, r)
        drain_s(pm, pm, rm)
        drain_idx(m + 1, p1)
        fire_idx(m + 2, p2)
        fire_g(p1, p1)
        return 0

    lax.fori_loop(1, NSUP - 2, step, 0)

    m = NSUP - 2
    p, pm, p1 = m % NIB, (m - 1) % NIB, (m + 1) % NIB
    r, rm = m % 2, (m - 1) % 2
    drain_g(p, p)
    fire_s(p, p, r)
    drain_s(pm, pm, rm)
    drain_idx(m + 1, p1)
    fire_g(p1, p1)

    m = NSUP - 1
    p, pm = m % NIB, (m - 1) % NIB
    r, rm = m % 2, (m - 1) % 2
    drain_g(p, p)
    fire_s(p, p, r)
    drain_s(pm, pm, rm)
    drain_s(p, p, r)

    plsc.subcore_barrier()
    pltpu.sync_copy(
        acc_sh.at[pl.ds(base, SLP)],
        out_hbm.at[cid, pl.ds(base, SLP)],
    )


BR = 1280
GRID = NP // BR

_row_spec = pl.BlockSpec((BR, D), lambda i: (i, 0))
_col_spec = pl.BlockSpec((BR, 1), lambda i: (i, 0))
_w_spec = pl.BlockSpec((D, D), lambda i: (0, 0))
_b_spec = pl.BlockSpec((1, D), lambda i: (0, 0))


def _scale_matmul_body(d0_ref, d1_ref, x_ref, w_ref, g_ref):
    s = lax.rsqrt(d0_ref[...] + d1_ref[...] + 1.0)
    g_ref[...] = jnp.dot(x_ref[...], w_ref[...],
                         preferred_element_type=jnp.float32) * s


def _scale_matmul(d0, d1, x, w):
    return pl.pallas_call(
        _scale_matmul_body,
        out_shape=jax.ShapeDtypeStruct((NP, D), jnp.float32),
        grid=(GRID,),
        in_specs=[_col_spec, _col_spec, _row_spec, _w_spec],
        out_specs=_row_spec,
    )(d0, d1, x, w)


def _mid_body(d0_ref, d1_ref, a0_ref, a1_ref, g_ref, b_ref, w_ref, o_ref):
    s = lax.rsqrt(d0_ref[...] + d1_ref[...] + 1.0)
    pre = s * (a0_ref[...] + a1_ref[...] + g_ref[...]) + b_ref[...]
    h = jnp.maximum(pre, 0.0)
    o_ref[...] = jnp.dot(h, w_ref[...], preferred_element_type=jnp.float32) * s


def _mid(d0, d1, a0, a1, g, b, w):
    return pl.pallas_call(
        _mid_body,
        out_shape=jax.ShapeDtypeStruct((NP, D), jnp.float32),
        grid=(GRID,),
        in_specs=[_col_spec, _col_spec, _row_spec, _row_spec, _row_spec,
                  _b_spec, _w_spec],
        out_specs=_row_spec,
    )(d0, d1, a0, a1, g, b, w)


def _final_body(d0_ref, d1_ref, a0_ref, a1_ref, g_ref, b_ref, o_ref):
    s = lax.rsqrt(d0_ref[...] + d1_ref[...] + 1.0)
    o_ref[...] = s * (a0_ref[...] + a1_ref[...] + g_ref[...]) + b_ref[...]


def _final(d0, d1, a0, a1, g, b):
    return pl.pallas_call(
        _final_body,
        out_shape=jax.ShapeDtypeStruct((NP, D), jnp.float32),
        grid=(GRID,),
        in_specs=[_col_spec, _col_spec, _row_spec, _row_spec, _row_spec,
                  _b_spec],
        out_specs=_row_spec,
    )(d0, d1, a0, a1, g, b)


def kernel(x, edge_index, W1, b1, W2, b2):
    src = edge_index[0].reshape(NW, NSUP, CH)
    dst_flat = edge_index[1]
    dst = dst_flat.reshape(NW, NSUP, CH)
    zrows = jnp.zeros((SLP, D), jnp.float32)
    x_pad = jnp.pad(x, ((0, NP - N), (0, 0)))
    b1r = b1.reshape(1, D)
    b2r = b2.reshape(1, D)

    deg2 = _deg_hist(dst_flat)
    d0 = deg2[0].reshape(NP, 1)
    d1 = deg2[1].reshape(NP, 1)

    g1 = _scale_matmul(d0, d1, x_pad, W1)
    acc1 = _edge_scatter(src, dst, g1, zrows)
    g2 = _mid(d0, d1, acc1[0], acc1[1], g1, b1r, W2)
    acc2 = _edge_scatter(src, dst, g2, zrows)
    out = _final(d0, d1, acc2[0], acc2[1], g2, b2r)
    return out[:N]


# speedup vs baseline: 1.0641x; 1.0641x over previous
import functools

import jax
import jax.numpy as jnp
from jax import lax
from jax.experimental import pallas as pl
from jax.experimental.pallas import tpu as pltpu
from jax.experimental.pallas import tpu_sc as plsc

N = 10000
E = 320000
D = 128
NP = 10240
NC = 2
NS = 16
NW = NC * NS
EPW = E // NW
CH = 125
IB = 4
NBLK = EPW // (IB * CH)
SLP = NP // NS

_MESH = plsc.VectorSubcoreMesh(core_axis_name="c", subcore_axis_name="s")
_SC_PARAMS = pltpu.CompilerParams(needs_layout_passes=False)


@functools.partial(
    pl.kernel,
    out_type=jax.ShapeDtypeStruct((NC, NP), jnp.float32),
    mesh=_MESH,
    compiler_params=_SC_PARAMS,
    scratch_types=[
        pltpu.VMEM((EPW,), jnp.int32),
        pltpu.VMEM((NP,), jnp.float32),
        pltpu.VMEM((NS, SLP), jnp.float32),
        pltpu.VMEM((SLP,), jnp.float32),
        pltpu.VMEM_SHARED((NS, NP), jnp.float32),
    ],
)
def _deg_hist(dst_hbm, out_hbm, dsts, hist, buf, comb, hist_all):
    cid = lax.axis_index("c")
    sid = lax.axis_index("s")
    wid = sid * NC + cid
    z16 = jnp.zeros((16,), jnp.float32)
    ones16 = jnp.ones((16,), jnp.float32)

    def zloop(i, _):
        hist[pl.ds(i * 16, 16)] = z16
        return 0

    lax.fori_loop(0, NP // 16, zloop, 0)
    pltpu.sync_copy(dst_hbm.at[pl.ds(wid * EPW, EPW)], dsts)

    def hloop(i, _):
        idx = dsts[pl.ds(i * 16, 16)]
        plsc.addupdate_scatter(hist, [idx], ones16)
        return 0

    lax.fori_loop(0, EPW // 16, hloop, 0)
    pltpu.sync_copy(hist, hist_all.at[sid])
    plsc.subcore_barrier()
    pltpu.sync_copy(hist_all.at[pl.ds(0, NS), pl.ds(sid * SLP, SLP)], buf)

    def cloop(k, _):
        v = buf[0, pl.ds(k * 16, 16)]
        for r in range(1, NS):
            v = v + buf[r, pl.ds(k * 16, 16)]
        comb[pl.ds(k * 16, 16)] = v
        return 0

    lax.fori_loop(0, SLP // 16, cloop, 0)
    pltpu.sync_copy(comb, out_hbm.at[cid, pl.ds(sid * SLP, SLP)])


@functools.partial(
    pl.kernel,
    out_type=jax.ShapeDtypeStruct((NC, NP, D), jnp.float32),
    mesh=_MESH,
    compiler_params=_SC_PARAMS,
    scratch_types=[
        pltpu.VMEM((2, IB, CH), jnp.int32),
        pltpu.VMEM((2, IB, CH), jnp.int32),
        pltpu.VMEM((2, CH, D), jnp.float32),
        pltpu.VMEM_SHARED((NP, D), jnp.float32),
        pltpu.SemaphoreType.DMA,
        pltpu.SemaphoreType.DMA,
        pltpu.SemaphoreType.DMA,
        pltpu.SemaphoreType.DMA,
    ],
)
def _edge_scatter(src_hbm, dst_hbm, g_hbm, z_hbm, out_hbm, sidx, didx, rows,
                  acc_sh, gsem, ssem0, ssem1, isem):
    cid = lax.axis_index("c")
    sid = lax.axis_index("s")
    wid = sid * NC + cid
    base = sid * SLP
    ssems = (ssem0, ssem1)

    def fire_g(slot, i, buf):
        pltpu.async_copy(g_hbm.at[sidx.at[slot, i]], rows.at[buf], gsem)

    def drain_g(slot, i, buf):
        pltpu.make_async_copy(g_hbm.at[sidx.at[slot, i]], rows.at[buf],
                              gsem).wait()

    def fire_s(slot, i, buf, r):
        pltpu.async_copy(rows.at[buf], acc_sh.at[didx.at[slot, i]], ssems[r],
                         add=True)

    def drain_s(slot, i, buf, r):
        pltpu.make_async_copy(rows.at[buf], acc_sh.at[didx.at[slot, i]],
                              ssems[r]).wait()

    def fire_idx(nb, slot):
        pltpu.async_copy(src_hbm.at[wid, nb], sidx.at[slot], isem)
        pltpu.async_copy(dst_hbm.at[wid, nb], didx.at[slot], isem)

    def drain_idx(nb, slot):
        pltpu.make_async_copy(src_hbm.at[wid, nb], sidx.at[slot], isem).wait()
        pltpu.make_async_copy(dst_hbm.at[wid, nb], didx.at[slot], isem).wait()

    pltpu.sync_copy(src_hbm.at[wid, 0], sidx.at[0])
    pltpu.sync_copy(dst_hbm.at[wid, 0], didx.at[0])
    fire_g(0, 0, 0)
    pltpu.sync_copy(z_hbm, acc_sh.at[pl.ds(base, SLP)])
    plsc.subcore_barrier()

    drain_g(0, 0, 0); fire_s(0, 0, 0, 0); fire_idx(1, 1); fire_g(0, 1, 1)
    drain_g(0, 1, 1); fire_s(0, 1, 1, 1); drain_s(0, 0, 0, 0); fire_g(0, 2, 0)
    drain_g(0, 2, 0); fire_s(0, 2, 0, 0); drain_s(0, 1, 1, 1)
    drain_idx(1, 1)
    fire_g(0, 3, 1)
    drain_g(0, 3, 1); fire_s(0, 3, 1, 1); drain_s(0, 2, 0, 0); fire_g(1, 0, 0)

    def blk_step(nb, _):
        q = lax.rem(nb, 2)
        q1 = lax.rem(nb + 1, 2)
        drain_g(q, 0, 0); fire_s(q, 0, 0, 0); drain_s(q1, 3, 1, 1)
        fire_idx(nb + 1, q1)
        fire_g(q, 1, 1)
        drain_g(q, 1, 1); fire_s(q, 1, 1, 1); drain_s(q, 0, 0, 0)
        fire_g(q, 2, 0)
        drain_g(q, 2, 0); fire_s(q, 2, 0, 0); drain_s(q, 1, 1, 1)
        drain_idx(nb + 1, q1)
        fire_g(q, 3, 1)
        drain_g(q, 3, 1); fire_s(q, 3, 1, 1); drain_s(q, 2, 0, 0)
        fire_g(q1, 0, 0)
        return 0

    lax.fori_loop(1, NBLK - 1, blk_step, 0)

    q = (NBLK - 1) % 2
    q1 = (NBLK) % 2
    drain_g(q, 0, 0); fire_s(q, 0, 0, 0); drain_s(q1, 3, 1, 1); fire_g(q, 1, 1)
    drain_g(q, 1, 1); fire_s(q, 1, 1, 1); drain_s(q, 0, 0, 0); fire_g(q, 2, 0)
    drain_g(q, 2, 0); fire_s(q, 2, 0, 0); drain_s(q, 1, 1, 1); fire_g(q, 3, 1)
    drain_g(q, 3, 1); fire_s(q, 3, 1, 1); drain_s(q, 2, 0, 0)
    drain_s(q, 3, 1, 1)

    plsc.subcore_barrier()
    pltpu.sync_copy(
        acc_sh.at[pl.ds(base, SLP)],
        out_hbm.at[cid, pl.ds(base, SLP)],
    )


BR = 1280
GRID = NP // BR

_row_spec = pl.BlockSpec((BR, D), lambda i: (i, 0))
_col_spec = pl.BlockSpec((BR, 1), lambda i: (i, 0))
_w_spec = pl.BlockSpec((D, D), lambda i: (0, 0))
_b_spec = pl.BlockSpec((1, D), lambda i: (0, 0))


def _scale_matmul_body(d0_ref, d1_ref, x_ref, w_ref, g_ref):
    s = lax.rsqrt(d0_ref[...] + d1_ref[...] + 1.0)
    g_ref[...] = jnp.dot(x_ref[...], w_ref[...],
                         preferred_element_type=jnp.float32) * s


def _scale_matmul(d0, d1, x, w):
    return pl.pallas_call(
        _scale_matmul_body,
        out_shape=jax.ShapeDtypeStruct((NP, D), jnp.float32),
        grid=(GRID,),
        in_specs=[_col_spec, _col_spec, _row_spec, _w_spec],
        out_specs=_row_spec,
    )(d0, d1, x, w)


def _mid_body(d0_ref, d1_ref, a0_ref, a1_ref, g_ref, b_ref, w_ref, o_ref):
    s = lax.rsqrt(d0_ref[...] + d1_ref[...] + 1.0)
    pre = s * (a0_ref[...] + a1_ref[...] + g_ref[...]) + b_ref[...]
    h = jnp.maximum(pre, 0.0)
    o_ref[...] = jnp.dot(h, w_ref[...], preferred_element_type=jnp.float32) * s


def _mid(d0, d1, a0, a1, g, b, w):
    return pl.pallas_call(
        _mid_body,
        out_shape=jax.ShapeDtypeStruct((NP, D), jnp.float32),
        grid=(GRID,),
        in_specs=[_col_spec, _col_spec, _row_spec, _row_spec, _row_spec,
                  _b_spec, _w_spec],
        out_specs=_row_spec,
    )(d0, d1, a0, a1, g, b, w)


def _final_body(d0_ref, d1_ref, a0_ref, a1_ref, g_ref, b_ref, o_ref):
    s = lax.rsqrt(d0_ref[...] + d1_ref[...] + 1.0)
    o_ref[...] = s * (a0_ref[...] + a1_ref[...] + g_ref[...]) + b_ref[...]


def _final(d0, d1, a0, a1, g, b):
    return pl.pallas_call(
        _final_body,
        out_shape=jax.ShapeDtypeStruct((NP, D), jnp.float32),
        grid=(GRID,),
        in_specs=[_col_spec, _col_spec, _row_spec, _row_spec, _row_spec,
                  _b_spec],
        out_specs=_row_spec,
    )(d0, d1, a0, a1, g, b)


def kernel(x, edge_index, W1, b1, W2, b2):
    src = edge_index[0].reshape(NW, NBLK, IB, CH)
    dst_flat = edge_index[1]
    dst = dst_flat.reshape(NW, NBLK, IB, CH)
    zrows = jnp.zeros((SLP, D), jnp.float32)
    x_pad = jnp.pad(x, ((0, NP - N), (0, 0)))
    b1r = b1.reshape(1, D)
    b2r = b2.reshape(1, D)

    deg2 = _deg_hist(dst_flat)
    d0 = deg2[0].reshape(NP, 1)
    d1 = deg2[1].reshape(NP, 1)

    g1 = _scale_matmul(d0, d1, x_pad, W1)
    acc1 = _edge_scatter(src, dst, g1, zrows)
    g2 = _mid(d0, d1, acc1[0], acc1[1], g1, b1r, W2)
    acc2 = _edge_scatter(src, dst, g2, zrows)
    out = _final(d0, d1, acc2[0], acc2[1], g2, b2r)
    return out[:N]


# speedup vs baseline: 1.1266x; 1.0588x over previous
import functools

import jax
import jax.numpy as jnp
from jax import lax
from jax.experimental import pallas as pl
from jax.experimental.pallas import tpu as pltpu
from jax.experimental.pallas import tpu_sc as plsc

N = 10000
E = 320000
D = 128
NP = 10240
NC = 2
NS = 16
NW = NC * NS
EPW = E // NW
CH = 128
NCH = EPW // CH
CT = EPW - NCH * CH
SLP = NP // NS

_MESH = plsc.VectorSubcoreMesh(core_axis_name="c", subcore_axis_name="s")
_SC_PARAMS = pltpu.CompilerParams(needs_layout_passes=False)


@functools.partial(
    pl.kernel,
    out_type=jax.ShapeDtypeStruct((NC, NP), jnp.float32),
    mesh=_MESH,
    compiler_params=_SC_PARAMS,
    scratch_types=[
        pltpu.VMEM((EPW,), jnp.int32),
        pltpu.VMEM((NP,), jnp.float32),
        pltpu.VMEM((NS, SLP), jnp.float32),
        pltpu.VMEM((SLP,), jnp.float32),
        pltpu.VMEM_SHARED((NS, NP), jnp.float32),
    ],
)
def _deg_hist(dst_hbm, out_hbm, dsts, hist, buf, comb, hist_all):
    cid = lax.axis_index("c")
    sid = lax.axis_index("s")
    wid = sid * NC + cid
    z16 = jnp.zeros((16,), jnp.float32)
    ones16 = jnp.ones((16,), jnp.float32)

    def zloop(i, _):
        hist[pl.ds(i * 16, 16)] = z16
        return 0

    lax.fori_loop(0, NP // 16, zloop, 0)
    pltpu.sync_copy(dst_hbm.at[pl.ds(wid * EPW, EPW)], dsts)

    def hloop(i, _):
        idx = dsts[pl.ds(i * 16, 16)]
        plsc.addupdate_scatter(hist, [idx], ones16)
        return 0

    lax.fori_loop(0, EPW // 16, hloop, 0)
    pltpu.sync_copy(hist, hist_all.at[sid])
    plsc.subcore_barrier()
    pltpu.sync_copy(hist_all.at[pl.ds(0, NS), pl.ds(sid * SLP, SLP)], buf)

    def cloop(k, _):
        v = buf[0, pl.ds(k * 16, 16)]
        for r in range(1, NS):
            v = v + buf[r, pl.ds(k * 16, 16)]
        comb[pl.ds(k * 16, 16)] = v
        return 0

    lax.fori_loop(0, SLP // 16, cloop, 0)
    pltpu.sync_copy(comb, out_hbm.at[cid, pl.ds(sid * SLP, SLP)])


@functools.partial(
    pl.kernel,
    out_type=jax.ShapeDtypeStruct((NC, NP, D), jnp.float32),
    mesh=_MESH,
    compiler_params=_SC_PARAMS,
    scratch_types=[
        pltpu.VMEM((3, CH), jnp.int32),
        pltpu.VMEM((3, CH), jnp.int32),
        pltpu.VMEM((2, CH, D), jnp.float32),
        pltpu.VMEM((CT,), jnp.int32),
        pltpu.VMEM((CT,), jnp.int32),
        pltpu.VMEM((CT, D), jnp.float32),
        pltpu.VMEM_SHARED((NP, D), jnp.float32),
        pltpu.SemaphoreType.DMA,
        pltpu.SemaphoreType.DMA,
        pltpu.SemaphoreType.DMA,
        pltpu.SemaphoreType.DMA,
    ],
)
def _edge_scatter(src_hbm, dst_hbm, g_hbm, z_hbm, out_hbm, sidx, didx, rows,
                  sidx_t, didx_t, trows, acc_sh, gsem, ssem0, ssem1, isem):
    cid = lax.axis_index("c")
    sid = lax.axis_index("s")
    wid = sid * NC + cid
    base = sid * SLP
    ebase = wid * EPW
    ssems = (ssem0, ssem1)

    def fire_g(m, p, buf):
        pltpu.async_copy(g_hbm.at[sidx.at[p]], rows.at[buf], gsem)

    def drain_g(m, p, buf):
        pltpu.make_async_copy(g_hbm.at[sidx.at[p]], rows.at[buf],
                              gsem).wait()

    def fire_s(m, p, buf, r):
        pltpu.async_copy(rows.at[buf], acc_sh.at[didx.at[p]], ssems[r],
                         add=True)

    def drain_s(m, p, buf, r):
        pltpu.make_async_copy(rows.at[buf], acc_sh.at[didx.at[p]],
                              ssems[r]).wait()

    def fire_idx(m, p):
        pltpu.async_copy(src_hbm.at[pl.ds(ebase + m * CH, CH)],
                         sidx.at[p], isem)
        pltpu.async_copy(dst_hbm.at[pl.ds(ebase + m * CH, CH)],
                         didx.at[p], isem)

    def drain_idx(m, p):
        pltpu.make_async_copy(src_hbm.at[pl.ds(ebase + m * CH, CH)],
                              sidx.at[p], isem).wait()
        pltpu.make_async_copy(dst_hbm.at[pl.ds(ebase + m * CH, CH)],
                              didx.at[p], isem).wait()

    pltpu.sync_copy(src_hbm.at[pl.ds(ebase, CH)], sidx.at[0])
    pltpu.sync_copy(dst_hbm.at[pl.ds(ebase, CH)], didx.at[0])
    fire_idx(1, 1)
    fire_g(0, 0, 0)
    pltpu.sync_copy(z_hbm, acc_sh.at[pl.ds(base, SLP)])
    plsc.subcore_barrier()

    drain_g(0, 0, 0); fire_s(0, 0, 0, 0)
    drain_idx(1, 1); fire_idx(2, 2); fire_g(1, 1, 1)
    drain_g(1, 1, 1); fire_s(1, 1, 1, 1); drain_s(0, 0, 0, 0)
    drain_idx(2, 2); fire_idx(3, 0); fire_g(2, 2, 0)

    def pair(t, _):
        m0 = 2 * t
        p0 = lax.rem(m0, 3)
        p1 = lax.rem(m0 + 1, 3)
        p2 = lax.rem(m0 + 2, 3)
        p3 = lax.rem(m0 + 3, 3)
        drain_g(m0, p0, 0); fire_s(m0, p0, 0, 0); drain_s(m0 - 1, p2, 1, 1)
        drain_idx(m0 + 1, p1); fire_idx(m0 + 2, p2); fire_g(m0 + 1, p1, 1)
        drain_g(m0 + 1, p1, 1); fire_s(m0 + 1, p1, 1, 1)
        drain_s(m0, p0, 0, 0)
        drain_idx(m0 + 2, p2); fire_idx(m0 + 3, p0); fire_g(m0 + 2, p2, 0)
        return 0

    lax.fori_loop(1, NCH // 2 - 1, pair, 0)

    m = NCH - 2
    p0, p1, p2 = m % 3, (m + 1) % 3, (m + 2) % 3
    drain_g(m, p0, 0); fire_s(m, p0, 0, 0); drain_s(m - 1, p2, 1, 1)
    drain_idx(m + 1, p1); fire_g(m + 1, p1, 1)
    m = NCH - 1
    drain_g(m, p1, 1); fire_s(m, p1, 1, 1); drain_s(m - 1, p0, 0, 0)

    toff = ebase + NCH * CH
    pltpu.sync_copy(src_hbm.at[pl.ds(toff, CT)], sidx_t)
    pltpu.sync_copy(dst_hbm.at[pl.ds(toff, CT)], didx_t)
    pltpu.async_copy(g_hbm.at[sidx_t], trows, gsem).wait()
    pltpu.sync_copy(trows, acc_sh.at[didx_t], add=True)
    drain_s(NCH - 1, p1, 1, 1)

    plsc.subcore_barrier()
    pltpu.sync_copy(
        acc_sh.at[pl.ds(base, SLP)],
        out_hbm.at[cid, pl.ds(base, SLP)],
    )


BR = 2000
GRID = N // BR

_row_spec = pl.BlockSpec((BR, D), lambda i: (i, 0))
_col_spec = pl.BlockSpec((BR, 1), lambda i: (i, 0))
_w_spec = pl.BlockSpec((D, D), lambda i: (0, 0))
_b_spec = pl.BlockSpec((1, D), lambda i: (0, 0))
_a0_spec = pl.BlockSpec((1, BR, D), lambda i: (0, i, 0))
_a1_spec = pl.BlockSpec((1, BR, D), lambda i: (1, i, 0))


def _scale_matmul_body(d0_ref, d1_ref, x_ref, w_ref, g_ref):
    s = lax.rsqrt(d0_ref[...] + d1_ref[...] + 1.0)
    g_ref[...] = jnp.dot(x_ref[...], w_ref[...],
                         preferred_element_type=jnp.float32) * s


def _scale_matmul(d0, d1, x, w):
    return pl.pallas_call(
        _scale_matmul_body,
        out_shape=jax.ShapeDtypeStruct((N, D), jnp.float32),
        grid=(GRID,),
        in_specs=[_col_spec, _col_spec, _row_spec, _w_spec],
        out_specs=_row_spec,
    )(d0, d1, x, w)


def _mid_body(d0_ref, d1_ref, a0_ref, a1_ref, g_ref, b_ref, w_ref, o_ref):
    s = lax.rsqrt(d0_ref[...] + d1_ref[...] + 1.0)
    pre = s * (a0_ref[0] + a1_ref[0] + g_ref[...]) + b_ref[...]
    h = jnp.maximum(pre, 0.0)
    o_ref[...] = jnp.dot(h, w_ref[...], preferred_element_type=jnp.float32) * s


def _mid(d0, d1, acc, g, b, w):
    return pl.pallas_call(
        _mid_body,
        out_shape=jax.ShapeDtypeStruct((N, D), jnp.float32),
        grid=(GRID,),
        in_specs=[_col_spec, _col_spec, _a0_spec, _a1_spec, _row_spec,
                  _b_spec, _w_spec],
        out_specs=_row_spec,
    )(d0, d1, acc, acc, g, b, w)


def _final_body(d0_ref, d1_ref, a0_ref, a1_ref, g_ref, b_ref, o_ref):
    s = lax.rsqrt(d0_ref[...] + d1_ref[...] + 1.0)
    o_ref[...] = s * (a0_ref[0] + a1_ref[0] + g_ref[...]) + b_ref[...]


def _final(d0, d1, acc, g, b):
    return pl.pallas_call(
        _final_body,
        out_shape=jax.ShapeDtypeStruct((N, D), jnp.float32),
        grid=(GRID,),
        in_specs=[_col_spec, _col_spec, _a0_spec, _a1_spec, _row_spec,
                  _b_spec],
        out_specs=_row_spec,
    )(d0, d1, acc, acc, g, b)


def kernel(x, edge_index, W1, b1, W2, b2):
    src = edge_index[0]
    dst = edge_index[1]
    zrows = jnp.zeros((SLP, D), jnp.float32)
    b1r = b1.reshape(1, D)
    b2r = b2.reshape(1, D)

    deg2 = _deg_hist(dst)
    d0 = deg2[0].reshape(NP, 1)
    d1 = deg2[1].reshape(NP, 1)

    g1 = _scale_matmul(d0, d1, x, W1)
    acc1 = _edge_scatter(src, dst, g1, zrows)
    g2 = _mid(d0, d1, acc1, g1, b1r, W2)
    acc2 = _edge_scatter(src, dst, g2, zrows)
    out = _final(d0, d1, acc2, g2, b2r)
    return out


# speedup vs baseline: 1.1732x; 1.0414x over previous
import functools

import jax
import jax.numpy as jnp
from jax import lax
from jax.experimental import pallas as pl
from jax.experimental.pallas import tpu as pltpu
from jax.experimental.pallas import tpu_sc as plsc

N = 10000
E = 320000
D = 128
NP = 10240
NC = 2
NS = 16
NW = NC * NS
CH = 128
NCHT = E // CH
NCH = NCHT // NW
NXW = NCHT - NCH * NW
SLP = NP // NS

_MESH = plsc.VectorSubcoreMesh(core_axis_name="c", subcore_axis_name="s")
_SC_PARAMS = pltpu.CompilerParams(needs_layout_passes=False)


@functools.partial(
    pl.kernel,
    out_type=jax.ShapeDtypeStruct((NC, NP), jnp.float32),
    mesh=_MESH,
    compiler_params=_SC_PARAMS,
    scratch_types=[
        pltpu.VMEM((NCH, 2, CH), jnp.int32),
        pltpu.VMEM((2, CH), jnp.int32),
        pltpu.VMEM((NP,), jnp.float32),
        pltpu.VMEM((NS, SLP), jnp.float32),
        pltpu.VMEM((SLP,), jnp.float32),
        pltpu.VMEM_SHARED((NS, NP), jnp.float32),
        pltpu.SemaphoreType.DMA,
    ],
)
def _deg_hist(ei_hbm, out_hbm, chunks, xtra, hist, buf, comb, hist_all, sem):
    cid = lax.axis_index("c")
    sid = lax.axis_index("s")
    wid = sid * NC + cid
    z16 = jnp.zeros((16,), jnp.float32)
    ones16 = jnp.ones((16,), jnp.float32)

    def fire_chunk(m, _):
        off = (m * NW + wid) * CH
        pltpu.async_copy(ei_hbm.at[pl.ds(0, 2), pl.ds(off, CH)],
                         chunks.at[m], sem)
        return 0

    lax.fori_loop(0, NCH, fire_chunk, 0)

    def zloop(i, _):
        hist[pl.ds(i * 16, 16)] = z16
        return 0

    lax.fori_loop(0, NP // 16, zloop, 0)

    def drain_chunk(m, _):
        pltpu.make_async_copy(ei_hbm.at[pl.ds(0, 2), pl.ds(wid * CH, CH)],
                              chunks.at[m], sem).wait()
        return 0

    lax.fori_loop(0, NCH, drain_chunk, 0)

    def hloop(i, _):
        idx = chunks[i >> 3, 1, pl.ds((i & 7) * 16, 16)]
        plsc.addupdate_scatter(hist, [idx], ones16)
        return 0

    lax.fori_loop(0, NCH * CH // 16, hloop, 0)

    @pl.when(wid < NXW)
    def _():
        pltpu.sync_copy(
            ei_hbm.at[pl.ds(0, 2), pl.ds((NCH * NW + wid) * CH, CH)], xtra)

        def xloop(i, _):
            idx = xtra[1, pl.ds(i * 16, 16)]
            plsc.addupdate_scatter(hist, [idx], ones16)
            return 0

        lax.fori_loop(0, CH // 16, xloop, 0)
    pltpu.sync_copy(hist, hist_all.at[sid])
    plsc.subcore_barrier()
    pltpu.sync_copy(hist_all.at[pl.ds(0, NS), pl.ds(sid * SLP, SLP)], buf)

    def cloop(k, _):
        v = buf[0, pl.ds(k * 16, 16)]
        for r in range(1, NS):
            v = v + buf[r, pl.ds(k * 16, 16)]
        comb[pl.ds(k * 16, 16)] = v
        return 0

    lax.fori_loop(0, SLP // 16, cloop, 0)
    pltpu.sync_copy(comb, out_hbm.at[cid, pl.ds(sid * SLP, SLP)])


@functools.partial(
    pl.kernel,
    out_type=jax.ShapeDtypeStruct((NC, NP, D), jnp.float32),
    mesh=_MESH,
    compiler_params=_SC_PARAMS,
    scratch_types=[
        pltpu.VMEM((3, 2, CH), jnp.int32),
        pltpu.VMEM((2, CH, D), jnp.float32),
        pltpu.VMEM_SHARED((NP, D), jnp.float32),
        pltpu.SemaphoreType.DMA,
        pltpu.SemaphoreType.DMA,
        pltpu.SemaphoreType.DMA,
        pltpu.SemaphoreType.DMA,
    ],
)
def _edge_scatter(ei_hbm, g_hbm, z_hbm, out_hbm, iblk, rows,
                  acc_sh, gsem, ssem0, ssem1, isem):
    cid = lax.axis_index("c")
    sid = lax.axis_index("s")
    wid = sid * NC + cid
    base = sid * SLP
    ssems = (ssem0, ssem1)

    def fire_g(m, p, buf):
        pltpu.async_copy(g_hbm.at[iblk.at[p, 0]], rows.at[buf], gsem)

    def drain_g(m, p, buf):
        pltpu.make_async_copy(g_hbm.at[iblk.at[p, 0]], rows.at[buf],
                              gsem).wait()

    def fire_s(m, p, buf, r):
        pltpu.async_copy(rows.at[buf], acc_sh.at[iblk.at[p, 1]], ssems[r],
                         add=True)

    def drain_s(m, p, buf, r):
        pltpu.make_async_copy(rows.at[buf], acc_sh.at[iblk.at[p, 1]],
                              ssems[r]).wait()

    def fire_idx(m, p):
        pltpu.async_copy(
            ei_hbm.at[pl.ds(0, 2), pl.ds((m * NW + wid) * CH, CH)],
            iblk.at[p], isem)

    def drain_idx(m, p):
        pltpu.make_async_copy(
            ei_hbm.at[pl.ds(0, 2), pl.ds((m * NW + wid) * CH, CH)],
            iblk.at[p], isem).wait()

    pltpu.sync_copy(ei_hbm.at[pl.ds(0, 2), pl.ds(wid * CH, CH)], iblk.at[0])
    fire_idx(1, 1)
    fire_g(0, 0, 0)
    pltpu.sync_copy(z_hbm, acc_sh.at[pl.ds(base, SLP)])
    plsc.subcore_barrier()

    drain_g(0, 0, 0); fire_s(0, 0, 0, 0)
    drain_idx(1, 1); fire_idx(2, 2); fire_g(1, 1, 1)
    drain_g(1, 1, 1); fire_s(1, 1, 1, 1); drain_s(0, 0, 0, 0)
    drain_idx(2, 2); fire_idx(3, 0); fire_g(2, 2, 0)

    def pair(t, _):
        m0 = 2 * t
        p0 = lax.rem(m0, 3)
        p1 = lax.rem(m0 + 1, 3)
        p2 = lax.rem(m0 + 2, 3)
        p3 = lax.rem(m0 + 3, 3)
        drain_g(m0, p0, 0); fire_s(m0, p0, 0, 0); drain_s(m0 - 1, p2, 1, 1)
        drain_idx(m0 + 1, p1); fire_idx(m0 + 2, p2); fire_g(m0 + 1, p1, 1)
        drain_g(m0 + 1, p1, 1); fire_s(m0 + 1, p1, 1, 1)
        drain_s(m0, p0, 0, 0)
        drain_idx(m0 + 2, p2); fire_idx(m0 + 3, p0); fire_g(m0 + 2, p2, 0)
        return 0

    lax.fori_loop(1, NCH // 2 - 1, pair, 0)

    m = NCH - 2
    p0, p1, p2 = m % 3, (m + 1) % 3, (m + 2) % 3
    drain_g(m, p0, 0); fire_s(m, p0, 0, 0); drain_s(m - 1, p2, 1, 1)
    drain_idx(m + 1, p1); fire_g(m + 1, p1, 1)
    m = NCH - 1
    drain_g(m, p1, 1); fire_s(m, p1, 1, 1); drain_s(m - 1, p0, 0, 0)

    drain_s(NCH - 1, p1, 1, 1)

    @pl.when(wid < NXW)
    def _():
        pltpu.sync_copy(
            ei_hbm.at[pl.ds(0, 2), pl.ds((NCH * NW + wid) * CH, CH)],
            iblk.at[0])
        pltpu.async_copy(g_hbm.at[iblk.at[0, 0]], rows.at[0], gsem).wait()
        pltpu.sync_copy(rows.at[0], acc_sh.at[iblk.at[0, 1]], add=True)

    plsc.subcore_barrier()
    pltpu.sync_copy(
        acc_sh.at[pl.ds(base, SLP)],
        out_hbm.at[cid, pl.ds(base, SLP)],
    )


BR = 2000
GRID = N // BR

_row_spec = pl.BlockSpec((BR, D), lambda i: (i, 0))
_col_spec = pl.BlockSpec((BR, 1), lambda i: (i, 0))
_w_spec = pl.BlockSpec((D, D), lambda i: (0, 0))
_b_spec = pl.BlockSpec((1, D), lambda i: (0, 0))
_a0_spec = pl.BlockSpec((1, BR, D), lambda i: (0, i, 0))
_a1_spec = pl.BlockSpec((1, BR, D), lambda i: (1, i, 0))


def _scale_matmul_body(d0_ref, d1_ref, x_ref, w_ref, g_ref):
    s = lax.rsqrt(d0_ref[...] + d1_ref[...] + 1.0)
    g_ref[...] = jnp.dot(x_ref[...], w_ref[...],
                         preferred_element_type=jnp.float32) * s


def _scale_matmul(d0, d1, x, w):
    return pl.pallas_call(
        _scale_matmul_body,
        out_shape=jax.ShapeDtypeStruct((N, D), jnp.float32),
        grid=(GRID,),
        in_specs=[_col_spec, _col_spec, _row_spec, _w_spec],
        out_specs=_row_spec,
    )(d0, d1, x, w)


def _mid_body(d0_ref, d1_ref, a0_ref, a1_ref, g_ref, b_ref, w_ref, o_ref):
    s = lax.rsqrt(d0_ref[...] + d1_ref[...] + 1.0)
    pre = s * (a0_ref[0] + a1_ref[0] + g_ref[...]) + b_ref[...]
    h = jnp.maximum(pre, 0.0)
    o_ref[...] = jnp.dot(h, w_ref[...], preferred_element_type=jnp.float32) * s


def _mid(d0, d1, acc, g, b, w):
    return pl.pallas_call(
        _mid_body,
        out_shape=jax.ShapeDtypeStruct((N, D), jnp.float32),
        grid=(GRID,),
        in_specs=[_col_spec, _col_spec, _a0_spec, _a1_spec, _row_spec,
                  _b_spec, _w_spec],
        out_specs=_row_spec,
    )(d0, d1, acc, acc, g, b, w)


def _final_body(d0_ref, d1_ref, a0_ref, a1_ref, g_ref, b_ref, o_ref):
    s = lax.rsqrt(d0_ref[...] + d1_ref[...] + 1.0)
    o_ref[...] = s * (a0_ref[0] + a1_ref[0] + g_ref[...]) + b_ref[...]


def _final(d0, d1, acc, g, b):
    return pl.pallas_call(
        _final_body,
        out_shape=jax.ShapeDtypeStruct((N, D), jnp.float32),
        grid=(GRID,),
        in_specs=[_col_spec, _col_spec, _a0_spec, _a1_spec, _row_spec,
                  _b_spec],
        out_specs=_row_spec,
    )(d0, d1, acc, acc, g, b)


def kernel(x, edge_index, W1, b1, W2, b2):
    zrows = jnp.zeros((SLP, D), jnp.float32)
    b1r = b1.reshape(1, D)
    b2r = b2.reshape(1, D)

    deg2 = _deg_hist(edge_index)
    d0 = deg2[0].reshape(NP, 1)
    d1 = deg2[1].reshape(NP, 1)

    g1 = _scale_matmul(d0, d1, x, W1)
    acc1 = _edge_scatter(edge_index, g1, zrows)
    g2 = _mid(d0, d1, acc1, g1, b1r, W2)
    acc2 = _edge_scatter(edge_index, g2, zrows)
    out = _final(d0, d1, acc2, g2, b2r)
    return out
